# Initial kernel scaffold; baseline (speedup 1.0000x reference)
import jax
import jax.numpy as jnp
from jax.experimental import pallas as pl


def kernel(x, role_ids, W, b, A, Bm):
    raise NotImplementedError("write your pallas kernel here")


# speedup vs baseline: 4.6793x; 4.6793x over previous
import jax
import jax.numpy as jnp
from jax.experimental import pallas as pl

_NUM_EXPERTS = 8
_RANK = 16
_SCALING = 2.0
_ER = _NUM_EXPERTS * _RANK
_TB = 512


def _fused_body(role_ref, x_ref, wt_ref, b_ref, at_ref, ball_ref, o_ref):
    xb = x_ref[...].astype(jnp.bfloat16)
    base = jnp.dot(xb, wt_ref[...], preferred_element_type=jnp.float32)
    u = jnp.dot(xb, at_ref[...], preferred_element_type=jnp.float32)
    role = role_ref[0, 0, :]
    col_expert = jax.lax.broadcasted_iota(jnp.int32, (1, _ER), 1) // _RANK
    mask = (role[:, None] == col_expert).astype(jnp.float32)
    um = (u * mask).astype(jnp.bfloat16)
    lora = jnp.dot(um, ball_ref[...], preferred_element_type=jnp.float32)
    o_ref[...] = base + _SCALING * lora + b_ref[...]


def kernel(x, role_ids, W, b, A, Bm):
    Bsz, T, D = x.shape
    O = W.shape[0]
    N = Bsz * T
    G = N // _TB
    x_flat = x.reshape(N, D)
    role3 = role_ids.reshape(G, 1, _TB).astype(jnp.int32)
    wt = W.T.astype(jnp.bfloat16)
    at = A.reshape(_ER, D).T.astype(jnp.bfloat16)
    ball = Bm.transpose(0, 2, 1).reshape(_ER, O).astype(jnp.bfloat16)
    b2 = b.reshape(1, O)
    out = pl.pallas_call(
        _fused_body,
        grid=(G,),
        in_specs=[
            pl.BlockSpec((1, 1, _TB), lambda i: (i, 0, 0)),
            pl.BlockSpec((_TB, D), lambda i: (i, 0)),
            pl.BlockSpec((D, O), lambda i: (0, 0)),
            pl.BlockSpec((1, O), lambda i: (0, 0)),
            pl.BlockSpec((D, _ER), lambda i: (0, 0)),
            pl.BlockSpec((_ER, O), lambda i: (0, 0)),
        ],
        out_specs=pl.BlockSpec((_TB, O), lambda i: (i, 0)),
        out_shape=jax.ShapeDtypeStruct((N, O), jnp.float32),
    )(role3, x_flat, wt, b2, at, ball)
    return out.reshape(Bsz, T, O)
